# baseline (speedup 1.0000x reference)
import jax
import jax.numpy as jnp
from jax import lax
from jax.experimental import pallas as pl
from jax.experimental.pallas import tpu as pltpu
from jax.experimental.pallas import tpu_sc as plsc

D_MODEL = 128
D_VOCAB = 100000
N_LOOKUPS = 4096 * 50
NC, NS = 2, 16
NW = NC * NS
PER_W = N_LOOKUPS // NW
CHUNK = 128
N_CHUNKS = PER_W // CHUNK


def _gather_body(x_hbm, w_hbm, out_hbm, idx_v, rows_v, sem):
    wid = lax.axis_index("s") * NC + lax.axis_index("c")
    pltpu.sync_copy(x_hbm.at[wid], idx_v)
    base = wid * PER_W

    def chunk(c, carry):
        pltpu.async_copy(w_hbm.at[idx_v.at[c]], rows_v, sem).wait()
        pltpu.sync_copy(rows_v, out_hbm.at[pl.ds(base + c * CHUNK, CHUNK)])
        return carry

    lax.fori_loop(0, N_CHUNKS, chunk, 0)


_T_BLK = 1024


def _transpose_body(w_ref, o_ref):
    o_ref[...] = w_ref[...].T


def _transpose_tc(W_E):
    grid = (D_VOCAB + _T_BLK - 1) // _T_BLK
    return pl.pallas_call(
        _transpose_body,
        grid=(grid,),
        in_specs=[pl.BlockSpec((D_MODEL, _T_BLK), lambda i: (0, i))],
        out_specs=pl.BlockSpec((_T_BLK, D_MODEL), lambda i: (i, 0)),
        out_shape=jax.ShapeDtypeStruct((D_VOCAB, D_MODEL), jnp.float32),
    )(W_E)


def kernel(x, W_E):
    W_T = _transpose_tc(W_E)
    x_r = x.reshape(NW, N_CHUNKS, CHUNK).astype(jnp.int32)

    mesh = plsc.VectorSubcoreMesh(
        core_axis_name="c", subcore_axis_name="s",
        num_cores=NC, num_subcores=NS,
    )
    out = pl.kernel(
        _gather_body,
        out_type=jax.ShapeDtypeStruct((N_LOOKUPS, D_MODEL), jnp.float32),
        mesh=mesh,
        scratch_types=[
            pltpu.VMEM((N_CHUNKS, CHUNK), jnp.int32),
            pltpu.VMEM((CHUNK, D_MODEL), jnp.float32),
            pltpu.SemaphoreType.DMA,
        ],
    )(x_r, W_T)
    return out.reshape(4096, 50, D_MODEL)


# speedup vs baseline: 1.7250x; 1.7250x over previous
import jax
import jax.numpy as jnp
from jax import lax
from jax.experimental import pallas as pl
from jax.experimental.pallas import tpu as pltpu
from jax.experimental.pallas import tpu_sc as plsc

D_MODEL = 128
D_VOCAB = 100000
BATCH = 4096
N_CTX = 50
NC, NS = 2, 16
NW = NC * NS
B_PER_W = BATCH // NW


def _gather_body(x_hbm, w_hbm, out_hbm, idx_v, rows_v, sem):
    wid = lax.axis_index("s") * NC + lax.axis_index("c")
    pltpu.sync_copy(x_hbm.at[wid], idx_v)
    base = wid * B_PER_W

    def chunk(c, carry):
        pltpu.async_copy(w_hbm.at[idx_v.at[c]], rows_v, sem).wait()
        pltpu.sync_copy(rows_v, out_hbm.at[base + c])
        return carry

    lax.fori_loop(0, B_PER_W, chunk, 0)


def kernel(x, W_E):
    W_T = W_E.T
    x_r = x.reshape(NW, B_PER_W, N_CTX).astype(jnp.int32)

    mesh = plsc.VectorSubcoreMesh(
        core_axis_name="c", subcore_axis_name="s",
        num_cores=NC, num_subcores=NS,
    )
    out = pl.kernel(
        _gather_body,
        out_type=jax.ShapeDtypeStruct((BATCH, N_CTX, D_MODEL), jnp.float32),
        mesh=mesh,
        scratch_types=[
            pltpu.VMEM((B_PER_W, N_CTX), jnp.int32),
            pltpu.VMEM((N_CTX, D_MODEL), jnp.float32),
            pltpu.SemaphoreType.DMA,
        ],
    )(x_r, W_T)
    return out
